# baseline (speedup 1.0000x reference)
import jax
import jax.numpy as jnp
from jax import lax
from jax.experimental import pallas as pl
from jax.experimental.pallas import tpu as pltpu

_B = 4096
_D = 128
_MBLK = 24
_PBLK = 16
_NB = 2


def _body(step_ref,
          ti_in, te_in, tip_in, tep_in, idx2_in, z_in, emb_in,
          ti_out, te_out, tip_out, tep_out,
          big_sem, small_sem, slice_sem):
    s = step_ref[0]
    rows = _B // _D
    nbig = _MBLK + _NB * _PBLK

    def start_ref_blk(i, _):
        @pl.when(i != s)
        def _():
            pltpu.make_async_copy(
                te_in.at[pl.ds(i * _B, _B), :],
                te_out.at[pl.ds(i * _B, _B), :], big_sem).start()

        @pl.when(i == s)
        def _():
            pltpu.make_async_copy(
                z_in, te_out.at[pl.ds(i * _B, _B), :], big_sem).start()
        return 0

    lax.fori_loop(0, _MBLK, start_ref_blk, 0)

    ps = lax.rem(s, _PBLK)

    def start_pos_blk(k, _):
        b = k // _PBLK
        j = lax.rem(k, _PBLK)

        @pl.when(j != ps)
        def _():
            pltpu.make_async_copy(
                tep_in.at[b, pl.ds(j * _B, _B), :],
                tep_out.at[b, pl.ds(j * _B, _B), :], big_sem).start()

        @pl.when(j == ps)
        def _():
            pltpu.make_async_copy(
                emb_in.at[b],
                tep_out.at[b, pl.ds(j * _B, _B), :], big_sem).start()
        return 0

    lax.fori_loop(0, _NB * _PBLK, start_pos_blk, 0)

    ti_cp = pltpu.make_async_copy(ti_in, ti_out, small_sem)
    tip_cp = pltpu.make_async_copy(tip_in, tip_out, small_sem)
    ti_cp.start()
    tip_cp.start()
    ti_cp.wait()
    tip_cp.wait()
    ti_sl = pltpu.make_async_copy(
        idx2_in, ti_out.at[pl.ds(s * rows, rows), :], slice_sem)
    tip_sl = pltpu.make_async_copy(
        idx2_in, tip_out.at[pl.ds(ps * rows, rows), :], slice_sem)
    ti_sl.start()
    tip_sl.start()
    ti_sl.wait()
    tip_sl.wait()

    def drain(i, _):
        pltpu.make_async_copy(
            z_in, te_out.at[pl.ds(0, _B), :], big_sem).wait()
        return 0

    lax.fori_loop(0, nbig, drain, 0)


def kernel(train_indices_ref, train_embeddings_ref, train_indices_pos,
           train_embeddings_pos, indices, Z_ssps, embeddings, step_rel):
    M = train_embeddings_ref.shape[0]
    P = train_indices_pos.shape[0]
    step = jnp.asarray(step_rel, jnp.int32).reshape(1)

    ti2 = train_indices_ref.reshape(M // _D, _D)
    tip2 = train_indices_pos.reshape(P // _D, _D)
    idx2 = indices.reshape(_B // _D, _D)

    hbm = pl.BlockSpec(memory_space=pltpu.MemorySpace.HBM)

    out = pl.pallas_call(
        _body,
        in_specs=[pl.BlockSpec(memory_space=pltpu.MemorySpace.SMEM)] + [hbm] * 7,
        out_specs=[hbm] * 4,
        out_shape=[
            jax.ShapeDtypeStruct(ti2.shape, jnp.int32),
            jax.ShapeDtypeStruct((M, _D), jnp.float32),
            jax.ShapeDtypeStruct(tip2.shape, jnp.int32),
            jax.ShapeDtypeStruct((_NB, P, _D), jnp.float32),
        ],
        scratch_shapes=[
            pltpu.SemaphoreType.DMA,
            pltpu.SemaphoreType.DMA,
            pltpu.SemaphoreType.DMA,
        ],
    )(step, ti2, train_embeddings_ref, tip2, train_embeddings_pos, idx2,
      Z_ssps, embeddings)

    return (out[0].reshape(M), out[1], out[2].reshape(P), out[3])


# speedup vs baseline: 46.6514x; 46.6514x over previous
import jax
import jax.numpy as jnp
from jax import lax
from jax.experimental import pallas as pl
from jax.experimental.pallas import tpu as pltpu

_B = 4096
_D = 128
_BR = 8192
_MBLK = 12
_PBLK = 8
_NB = 2
_GRID = _NB * _PBLK


def _body(step_ref,
          ti_ref_in, te_ref_in, tip_in, tep_in, idx2_in, z_in, emb_in,
          ti_ref_out, te_ref_out, tip_out, tep_out):
    i = pl.program_id(0)
    s = step_ref[0]
    sblk, soff = s // 2, lax.rem(s, 2) * _B
    ps = lax.rem(s, _NB * _PBLK)
    pblk, poff = ps // 2, lax.rem(ps, 2) * _B

    @pl.when(i < _MBLK)
    def _():
        te_ref_out[...] = te_ref_in[...]

        @pl.when(i == sblk)
        def _():
            te_ref_out[pl.ds(soff, _B), :] = z_in[...]

    j = lax.rem(i, _PBLK)
    tep_out[...] = tep_in[...]

    @pl.when(j == pblk)
    def _():
        tep_out[0, pl.ds(poff, _B), :] = emb_in[0]

    @pl.when(i == 0)
    def _():
        rows = _B // _D
        ti_ref_out[...] = ti_ref_in[...]
        ti_ref_out[pl.ds(s * rows, rows), :] = idx2_in[...]
        tip_out[...] = tip_in[...]
        tip_out[pl.ds(lax.rem(s * rows, tip_out.shape[0]), rows), :] = idx2_in[...]


def kernel(train_indices_ref, train_embeddings_ref, train_indices_pos,
           train_embeddings_pos, indices, Z_ssps, embeddings, step_rel):
    M = train_embeddings_ref.shape[0]
    P = train_indices_pos.shape[0]
    step = jnp.asarray(step_rel, jnp.int32).reshape(1)

    ti2 = train_indices_ref.reshape(M // _D, _D)
    tip2 = train_indices_pos.reshape(P // _D, _D)
    idx2 = indices.reshape(_B // _D, _D)

    full = lambda shape: pl.BlockSpec(shape, lambda i, s: (0,) * len(shape))

    out = pl.pallas_call(
        _body,
        grid_spec=pltpu.PrefetchScalarGridSpec(
            num_scalar_prefetch=1,
            grid=(_GRID,),
            in_specs=[
                full(ti2.shape),
                pl.BlockSpec((_BR, _D),
                             lambda i, s: (jnp.minimum(i, _MBLK - 1), 0)),
                full(tip2.shape),
                pl.BlockSpec((1, _BR, _D),
                             lambda i, s: (i // _PBLK, lax.rem(i, _PBLK), 0)),
                full(idx2.shape),
                full((_B, _D)),
                pl.BlockSpec((1, _B, _D), lambda i, s: (i // _PBLK, 0, 0)),
            ],
            out_specs=[
                full(ti2.shape),
                pl.BlockSpec((_BR, _D),
                             lambda i, s: (jnp.minimum(i, _MBLK - 1), 0)),
                full(tip2.shape),
                pl.BlockSpec((1, _BR, _D),
                             lambda i, s: (i // _PBLK, lax.rem(i, _PBLK), 0)),
            ],
        ),
        out_shape=[
            jax.ShapeDtypeStruct(ti2.shape, jnp.int32),
            jax.ShapeDtypeStruct((M, _D), jnp.float32),
            jax.ShapeDtypeStruct(tip2.shape, jnp.int32),
            jax.ShapeDtypeStruct((_NB, P, _D), jnp.float32),
        ],
        compiler_params=pltpu.CompilerParams(
            dimension_semantics=("arbitrary",),
        ),
    )(step, ti2, train_embeddings_ref, tip2, train_embeddings_pos, idx2,
      Z_ssps, embeddings)

    return (out[0].reshape(M), out[1], out[2].reshape(P), out[3])
